# Initial kernel scaffold; baseline (speedup 1.0000x reference)
import jax
import jax.numpy as jnp
from jax.experimental import pallas as pl


def kernel(text_encodings, lm_head_weight):
    raise NotImplementedError("write your pallas kernel here")


# speedup vs baseline: 1.2743x; 1.2743x over previous
import functools

import jax
import jax.numpy as jnp
from jax import lax
from jax.experimental import pallas as pl
from jax.experimental.pallas import tpu as pltpu
from jax.experimental.pallas import tpu_sc as plsc

_VOCAB = 100000
_D = 128
_B = 4096 * 20

_INFO = plsc.get_sparse_core_info()
_NC = _INFO.num_cores
_NS = _INFO.num_subcores
_NW = _NC * _NS
_B_PER_W = _B // _NW
_CHUNK = 128
_N_CHUNKS = _B_PER_W // _CHUNK
_NBUF = 2


def _make_gather():
    mesh = plsc.VectorSubcoreMesh(core_axis_name="c", subcore_axis_name="s")

    @functools.partial(
        pl.kernel,
        mesh=mesh,
        out_type=jax.ShapeDtypeStruct((_B, _D), jnp.float32),
        scratch_types=[
            pltpu.VMEM((_N_CHUNKS, _CHUNK), jnp.int32),
            pltpu.VMEM((_NBUF, _CHUNK, _D), jnp.float32),
            pltpu.SemaphoreType.DMA,
            pltpu.SemaphoreType.DMA,
        ],
    )
    def gather_kernel(table_hbm, idx_hbm, out_hbm, idx_v, rows_v, sem0, sem1):
        wid = lax.axis_index("s") * _NC + lax.axis_index("c")
        base = wid * _B_PER_W
        pltpu.sync_copy(idx_hbm.at[wid], idx_v)

        sems = (sem0, sem1)

        def start(j, b):
            return pltpu.async_copy(
                table_hbm.at[idx_v.at[j]], rows_v.at[b], sems[b]
            )

        handles = [None] * _N_CHUNKS
        handles[0] = start(0, 0)
        for j in range(_N_CHUNKS):
            b = j % _NBUF
            if j + 1 < _N_CHUNKS:
                handles[j + 1] = start(j + 1, (j + 1) % _NBUF)
            handles[j].wait()
            pltpu.sync_copy(
                rows_v.at[b], out_hbm.at[pl.ds(base + j * _CHUNK, _CHUNK)]
            )

    return gather_kernel


_GATHER = _make_gather()


@jax.jit
def kernel(text_encodings, lm_head_weight):
    idx = text_encodings.astype(jnp.int32).reshape(_NW, _N_CHUNKS, _CHUNK)
    out = _GATHER(lm_head_weight, idx)
    return out.reshape(text_encodings.shape[0], text_encodings.shape[1], _D)


# speedup vs baseline: 3.2705x; 2.5666x over previous
import functools

import jax
import jax.numpy as jnp
from jax import lax
from jax.experimental import pallas as pl
from jax.experimental.pallas import tpu as pltpu
from jax.experimental.pallas import tpu_sc as plsc

_VOCAB = 100000
_D = 128
_B = 4096 * 20

_INFO = plsc.get_sparse_core_info()
_NC = _INFO.num_cores
_NS = _INFO.num_subcores
_NW = _NC * _NS
_B_PER_W = _B // _NW
_CHUNK = 128
_N_CHUNKS = _B_PER_W // _CHUNK
_NBUF = 2


def _make_gather():
    mesh = plsc.VectorSubcoreMesh(core_axis_name="c", subcore_axis_name="s")

    @functools.partial(
        pl.kernel,
        mesh=mesh,
        out_type=jax.ShapeDtypeStruct((_B, _D), jnp.float32),
        scratch_types=[
            pltpu.VMEM((_N_CHUNKS, _CHUNK), jnp.int32),
            pltpu.VMEM((_NBUF, _CHUNK, _D), jnp.float32),
            pltpu.SemaphoreType.DMA,
            pltpu.SemaphoreType.DMA,
        ],
    )
    def gather_kernel(table_hbm, idx_hbm, out_hbm, idx_v, rows_v, sem0, sem1):
        wid = lax.axis_index("s") * _NC + lax.axis_index("c")
        base = wid * _B_PER_W
        pltpu.sync_copy(idx_hbm.at[wid], idx_v)

        sems = (sem0, sem1)

        def start(j, b):
            return pltpu.async_copy(
                table_hbm.at[idx_v.at[j]], rows_v.at[b], sems[b]
            )

        handles = [None] * _N_CHUNKS
        handles[0] = start(0, 0)
        for j in range(_N_CHUNKS):
            b = j % _NBUF
            if j + 1 < _N_CHUNKS:
                handles[j + 1] = start(j + 1, (j + 1) % _NBUF)
            handles[j].wait()
            pltpu.sync_copy(
                rows_v.at[b], out_hbm.at[pl.ds(base + j * _CHUNK, _CHUNK)]
            )

    return gather_kernel


_GATHER = _make_gather()


@jax.jit
def kernel(text_encodings, lm_head_weight):
    b, s = text_encodings.shape
    idx = text_encodings.T.astype(jnp.int32).reshape(_NW, _N_CHUNKS, _CHUNK)
    out = _GATHER(lm_head_weight, idx)
    return out.reshape(s, b, _D).transpose(1, 0, 2)


# speedup vs baseline: 3.3514x; 1.0247x over previous
import functools

import jax
import jax.numpy as jnp
from jax import lax
from jax.experimental import pallas as pl
from jax.experimental.pallas import tpu as pltpu
from jax.experimental.pallas import tpu_sc as plsc

_VOCAB = 100000
_D = 128
_B = 4096 * 20

_INFO = plsc.get_sparse_core_info()
_NC = _INFO.num_cores
_NS = _INFO.num_subcores
_NW = _NC * _NS
_B_PER_W = _B // _NW
_CHUNK = 128
_N_CHUNKS = _B_PER_W // _CHUNK
_NBUF = 4
_AHEAD = 2


def _make_gather():
    mesh = plsc.VectorSubcoreMesh(core_axis_name="c", subcore_axis_name="s")

    @functools.partial(
        pl.kernel,
        mesh=mesh,
        out_type=jax.ShapeDtypeStruct((_B, _D), jnp.float32),
        scratch_types=[
            pltpu.VMEM((_N_CHUNKS, _CHUNK), jnp.int32),
            pltpu.VMEM((_NBUF, _CHUNK, _D), jnp.float32),
        ]
        + [pltpu.SemaphoreType.DMA] * (2 * _NBUF),
    )
    def gather_kernel(
        table_hbm, idx_hbm, out_hbm, idx_v, rows_v,
        g0, g1, g2, g3, w0, w1, w2, w3,
    ):
        wid = lax.axis_index("s") * _NC + lax.axis_index("c")
        base = wid * _B_PER_W
        pltpu.sync_copy(idx_hbm.at[wid], idx_v)

        gsems = (g0, g1, g2, g3)
        wsems = (w0, w1, w2, w3)

        def gather(j):
            return pltpu.async_copy(
                table_hbm.at[idx_v.at[j]], rows_v.at[j % _NBUF], gsems[j % _NBUF]
            )

        def write(j):
            return pltpu.async_copy(
                rows_v.at[j % _NBUF],
                out_hbm.at[pl.ds(base + j * _CHUNK, _CHUNK)],
                wsems[j % _NBUF],
            )

        gh = [None] * _N_CHUNKS
        wh = [None] * _N_CHUNKS
        for j in range(_AHEAD):
            gh[j] = gather(j)
        for j in range(_N_CHUNKS):
            nxt = j + _AHEAD
            if nxt < _N_CHUNKS:
                prev_write = nxt - _NBUF
                if prev_write >= 0:
                    wh[prev_write].wait()
                gh[nxt] = gather(nxt)
            gh[j].wait()
            wh[j] = write(j)
        for j in range(max(0, _N_CHUNKS - _NBUF), _N_CHUNKS):
            if wh[j] is not None:
                wh[j].wait()

    return gather_kernel


_GATHER = _make_gather()


@jax.jit
def kernel(text_encodings, lm_head_weight):
    b, s = text_encodings.shape
    idx = text_encodings.T.astype(jnp.int32).reshape(_NW, _N_CHUNKS, _CHUNK)
    out = _GATHER(lm_head_weight, idx)
    return out.reshape(s, b, _D).transpose(1, 0, 2)


# speedup vs baseline: 3.4000x; 1.0145x over previous
import functools

import jax
import jax.numpy as jnp
from jax import lax
from jax.experimental import pallas as pl
from jax.experimental.pallas import tpu as pltpu
from jax.experimental.pallas import tpu_sc as plsc

_VOCAB = 100000
_D = 128
_B = 4096 * 20

_INFO = plsc.get_sparse_core_info()
_NC = _INFO.num_cores
_NS = _INFO.num_subcores
_NW = _NC * _NS
_B_PER_W = _B // _NW
_CHUNK = 128
_N_CHUNKS = _B_PER_W // _CHUNK
_GROUP = 2
_N_MACRO = _N_CHUNKS // _GROUP
_NBUF = 3
_AHEAD = 2


def _make_gather():
    mesh = plsc.VectorSubcoreMesh(core_axis_name="c", subcore_axis_name="s")

    @functools.partial(
        pl.kernel,
        mesh=mesh,
        out_type=jax.ShapeDtypeStruct((_B, _D), jnp.float32),
        scratch_types=[
            pltpu.VMEM((_N_CHUNKS, _CHUNK), jnp.int32),
            pltpu.VMEM((_NBUF, _GROUP * _CHUNK, _D), jnp.float32),
        ]
        + [pltpu.SemaphoreType.DMA] * (2 * _NBUF),
    )
    def gather_kernel(
        table_hbm, idx_hbm, out_hbm, idx_v, rows_v,
        g0, g1, g2, w0, w1, w2,
    ):
        wid = lax.axis_index("s") * _NC + lax.axis_index("c")
        base = wid * _B_PER_W
        pltpu.sync_copy(idx_hbm.at[wid], idx_v)

        gsems = (g0, g1, g2)
        wsems = (w0, w1, w2)

        def gather(m):
            b = m % _NBUF
            return [
                pltpu.async_copy(
                    table_hbm.at[idx_v.at[m * _GROUP + g]],
                    rows_v.at[b, pl.ds(g * _CHUNK, _CHUNK)],
                    gsems[b],
                )
                for g in range(_GROUP)
            ]

        def write(m):
            b = m % _NBUF
            return pltpu.async_copy(
                rows_v.at[b],
                out_hbm.at[pl.ds(base + m * _GROUP * _CHUNK, _GROUP * _CHUNK)],
                wsems[b],
            )

        gh = [None] * _N_MACRO
        wh = [None] * _N_MACRO
        for m in range(_AHEAD):
            gh[m] = gather(m)
        for m in range(_N_MACRO):
            nxt = m + _AHEAD
            if nxt < _N_MACRO:
                prev_write = nxt - _NBUF
                if prev_write >= 0:
                    wh[prev_write].wait()
                gh[nxt] = gather(nxt)
            for h in gh[m]:
                h.wait()
            wh[m] = write(m)
        for m in range(max(0, _N_MACRO - _NBUF), _N_MACRO):
            if wh[m] is not None:
                wh[m].wait()

    return gather_kernel


_GATHER = _make_gather()


@jax.jit
def kernel(text_encodings, lm_head_weight):
    b, s = text_encodings.shape
    idx = text_encodings.T.astype(jnp.int32).reshape(_NW, _N_CHUNKS, _CHUNK)
    out = _GATHER(lm_head_weight, idx)
    return out.reshape(s, b, _D).transpose(1, 0, 2)
